# Initial kernel scaffold; baseline (speedup 1.0000x reference)
import jax
import jax.numpy as jnp
from jax.experimental import pallas as pl


def kernel(x, edge_index, W1, b1, W2, b2, W3, b3, Wc, bc):
    raise NotImplementedError("write your pallas kernel here")


# speedup vs baseline: 14.0611x; 14.0611x over previous
import functools

import jax
import jax.numpy as jnp
from jax import lax
from jax.experimental import pallas as pl
from jax.experimental.pallas import tpu as pltpu
from jax.experimental.pallas import tpu_sc as plsc

NC = 2
NS = 16
NW = NC * NS
EC = 128
DW = 16
ZC = 208


def _row_split(n):
    rb = (n // NS) & ~7
    ex = n - NS * rb
    assert rb % ZC == 0 and ex % 8 == 0 and ex < ZC
    return rb, ex


def _rows_out(acc, out_hbm, c, s, n, copy):
    rb, ex = _row_split(n)
    base = pl.multiple_of(s * rb, 8)
    for k in range(rb // ZC):
        copy(base + k * ZC, ZC, False)
    if ex:
        @pl.when(s == NS - 1)
        def _():
            copy(n - ex, ex, True)


def _deg_sc(dst, n):
    e = dst.shape[0]
    ept = e // NW
    nfull, tail = ept // EC, ept % EC
    assert e % NW == 0 and tail % 8 == 0 and ept % 8 == 0

    mesh = plsc.VectorSubcoreMesh(core_axis_name="c", subcore_axis_name="s")

    def body(dst_hbm, out_hbm, acc, ones_v, ones_t, didx, didx_t, zbuf):
        c = lax.axis_index("c")
        s = lax.axis_index("s")
        wid = s * NC + c

        zero16 = jnp.zeros((16,), jnp.float32)
        one16 = jnp.ones((16,), jnp.float32)

        def fill(i, _):
            zbuf[i, :] = zero16
            return 0

        lax.fori_loop(0, ZC, fill, 0)

        def fill2(i, _):
            ones_v[i, :] = one16
            return 0

        lax.fori_loop(0, EC, fill2, 0)
        if tail:
            def fill3(i, _):
                ones_t[i, :] = one16
                return 0

            lax.fori_loop(0, tail, fill3, 0)

        def zcopy(off, cnt, is_tail):
            zsrc = zbuf.at[pl.ds(0, cnt)] if is_tail else zbuf
            pltpu.sync_copy(zsrc, acc.at[pl.ds(off, cnt)])

        _rows_out(acc, out_hbm, c, s, n, zcopy)
        plsc.subcore_barrier()

        eb = wid * ept

        def chunk(j, _):
            pltpu.sync_copy(dst_hbm.at[pl.ds(eb + j * EC, EC)], didx)
            pltpu.sync_copy(ones_v, acc.at[didx], add=True)
            return 0

        lax.fori_loop(0, nfull, chunk, 0)
        if tail:
            pltpu.sync_copy(dst_hbm.at[pl.ds(eb + nfull * EC, tail)], didx_t)
            pltpu.sync_copy(ones_t, acc.at[didx_t], add=True)

        plsc.subcore_barrier()

        def wcopy(off, cnt, is_tail):
            pltpu.sync_copy(acc.at[pl.ds(off, cnt)],
                            out_hbm.at[c, pl.ds(off, cnt)])

        _rows_out(acc, out_hbm, c, s, n, wcopy)

    scratch = [
        pltpu.VMEM_SHARED((n, DW), jnp.float32),
        pltpu.VMEM((EC, DW), jnp.float32),
        pltpu.VMEM((max(tail, 8), DW), jnp.float32),
        pltpu.VMEM((EC,), jnp.int32),
        pltpu.VMEM((max(tail, 8),), jnp.int32),
        pltpu.VMEM((ZC, DW), jnp.float32),
    ]
    return pl.kernel(
        body,
        jax.ShapeDtypeStruct((NC, n, DW), jnp.float32),
        mesh=mesh,
        scratch_types=scratch,
    )(dst)


def _agg_sc(hp, src, dst):
    n, d = hp.shape
    e = src.shape[0]
    ept = e // NW
    nfull, tail = ept // EC, ept % EC
    assert e % NW == 0 and d % 16 == 0
    assert tail % 8 == 0 and ept % 8 == 0

    mesh = plsc.VectorSubcoreMesh(core_axis_name="c", subcore_axis_name="s")

    def body(hp_hbm, src_hbm, dst_hbm, out_hbm,
             acc, rows, rows_t, sidx, didx, sidx_t, didx_t, zbuf, sem):
        c = lax.axis_index("c")
        s = lax.axis_index("s")
        wid = s * NC + c

        zero16 = jnp.zeros((16,), jnp.float32)

        def fill(i, _):
            for q in range(d // 16):
                zbuf[i, pl.ds(q * 16, 16)] = zero16
            return 0

        lax.fori_loop(0, ZC, fill, 0)

        def zcopy(off, cnt, is_tail):
            zsrc = zbuf.at[pl.ds(0, cnt)] if is_tail else zbuf
            pltpu.sync_copy(zsrc, acc.at[pl.ds(off, cnt)])

        _rows_out(acc, out_hbm, c, s, n, zcopy)
        plsc.subcore_barrier()

        eb = wid * ept

        def chunk(j, _):
            b0 = eb + j * EC
            pltpu.sync_copy(src_hbm.at[pl.ds(b0, EC)], sidx)
            pltpu.sync_copy(dst_hbm.at[pl.ds(b0, EC)], didx)
            pltpu.async_copy(hp_hbm.at[sidx], rows, sem).wait()
            pltpu.sync_copy(rows, acc.at[didx], add=True)
            return 0

        lax.fori_loop(0, nfull, chunk, 0)
        if tail:
            b0 = eb + nfull * EC
            pltpu.sync_copy(src_hbm.at[pl.ds(b0, tail)], sidx_t)
            pltpu.sync_copy(dst_hbm.at[pl.ds(b0, tail)], didx_t)
            pltpu.async_copy(hp_hbm.at[sidx_t], rows_t, sem).wait()
            pltpu.sync_copy(rows_t, acc.at[didx_t], add=True)

        plsc.subcore_barrier()

        def wcopy(off, cnt, is_tail):
            pltpu.sync_copy(acc.at[pl.ds(off, cnt)],
                            out_hbm.at[c, pl.ds(off, cnt)])

        _rows_out(acc, out_hbm, c, s, n, wcopy)

    scratch = [
        pltpu.VMEM_SHARED((n, d), jnp.float32),
        pltpu.VMEM((EC, d), jnp.float32),
        pltpu.VMEM((max(tail, 8), d), jnp.float32),
        pltpu.VMEM((EC,), jnp.int32),
        pltpu.VMEM((EC,), jnp.int32),
        pltpu.VMEM((max(tail, 8),), jnp.int32),
        pltpu.VMEM((max(tail, 8),), jnp.int32),
        pltpu.VMEM((ZC, d), jnp.float32),
        pltpu.SemaphoreType.DMA,
    ]
    return pl.kernel(
        body,
        jax.ShapeDtypeStruct((NC, n, d), jnp.float32),
        mesh=mesh,
        scratch_types=scratch,
    )(hp, src, dst)


_TC_R = 1000


def _dinv_of(g):
    deg = g[0, :, :1] + g[1, :, :1] + 1.0
    return lax.rsqrt(deg)


def _tc_first(degp, x, w):
    n, d = x.shape

    def body(g_ref, x_ref, w_ref, o_ref):
        dinv = _dinv_of(g_ref[...])
        o_ref[...] = dinv * jnp.dot(x_ref[...], w_ref[...],
                                    preferred_element_type=jnp.float32)

    return pl.pallas_call(
        body,
        grid=(n // _TC_R,),
        in_specs=[
            pl.BlockSpec((2, _TC_R, DW), lambda i: (0, i, 0)),
            pl.BlockSpec((_TC_R, d), lambda i: (i, 0)),
            pl.BlockSpec((d, w.shape[1]), lambda i: (0, 0)),
        ],
        out_specs=pl.BlockSpec((_TC_R, w.shape[1]), lambda i: (i, 0)),
        out_shape=jax.ShapeDtypeStruct((n, w.shape[1]), jnp.float32),
    )(degp, x, w)


def _tc_mid(degp, tmpp, hp, b, w):
    n, d = hp.shape

    def body(g_ref, t_ref, hp_ref, b_ref, w_ref, o_ref):
        dinv = _dinv_of(g_ref[...])
        t = t_ref[...]
        h = jnp.maximum(dinv * (t[0] + t[1] + hp_ref[...]) + b_ref[...], 0.0)
        o_ref[...] = dinv * jnp.dot(h, w_ref[...],
                                    preferred_element_type=jnp.float32)

    return pl.pallas_call(
        body,
        grid=(n // _TC_R,),
        in_specs=[
            pl.BlockSpec((2, _TC_R, DW), lambda i: (0, i, 0)),
            pl.BlockSpec((2, _TC_R, d), lambda i: (0, i, 0)),
            pl.BlockSpec((_TC_R, d), lambda i: (i, 0)),
            pl.BlockSpec((1, d), lambda i: (0, 0)),
            pl.BlockSpec((d, w.shape[1]), lambda i: (0, 0)),
        ],
        out_specs=pl.BlockSpec((_TC_R, w.shape[1]), lambda i: (i, 0)),
        out_shape=jax.ShapeDtypeStruct((n, w.shape[1]), jnp.float32),
    )(degp, tmpp, hp, b, w)


def _tc_last(degp, tmpp, hp, b, wc, bc):
    n, d = hp.shape
    dout = wc.shape[1]

    def body(g_ref, t_ref, hp_ref, b_ref, w_ref, bc_ref, o_ref):
        dinv = _dinv_of(g_ref[...])
        t = t_ref[...]
        h = jnp.maximum(dinv * (t[0] + t[1] + hp_ref[...]) + b_ref[...], 0.0)
        o_ref[...] = jnp.dot(h, w_ref[...],
                             preferred_element_type=jnp.float32) + bc_ref[...]

    return pl.pallas_call(
        body,
        grid=(n // _TC_R,),
        in_specs=[
            pl.BlockSpec((2, _TC_R, DW), lambda i: (0, i, 0)),
            pl.BlockSpec((2, _TC_R, d), lambda i: (0, i, 0)),
            pl.BlockSpec((_TC_R, d), lambda i: (i, 0)),
            pl.BlockSpec((1, d), lambda i: (0, 0)),
            pl.BlockSpec((d, dout), lambda i: (0, 0)),
            pl.BlockSpec((1, dout), lambda i: (0, 0)),
        ],
        out_specs=pl.BlockSpec((_TC_R, dout), lambda i: (i, 0)),
        out_shape=jax.ShapeDtypeStruct((n, dout), jnp.float32),
    )(degp, tmpp, hp, b, wc, bc)


def kernel(x, edge_index, W1, b1, W2, b2, W3, b3, Wc, bc):
    src = edge_index[0]
    dst = edge_index[1]
    n = x.shape[0]

    degp = _deg_sc(dst, n)
    hp = _tc_first(degp, x, W1)
    for (b, wn) in ((b1, W2), (b2, W3)):
        tmpp = _agg_sc(hp, src, dst)
        hp = _tc_mid(degp, tmpp, hp, b.reshape(1, -1), wn)
    tmpp = _agg_sc(hp, src, dst)
    return _tc_last(degp, tmpp, hp, b3.reshape(1, -1), Wc, bc.reshape(1, -1))


# speedup vs baseline: 25.5142x; 1.8145x over previous
import functools

import jax
import jax.numpy as jnp
from jax import lax
from jax.experimental import pallas as pl
from jax.experimental.pallas import tpu as pltpu
from jax.experimental.pallas import tpu_sc as plsc

NC = 2
NS = 16
NW = NC * NS
EC = 128
DW = 16
ZC = 208
ZB = 48


def _rows_out(acc, out_hbm, c, s, n, copy, chunk):
    rb = (n // NS) & ~7
    ex = n - NS * rb
    assert rb % chunk == 0 and ex % 8 == 0 and ex < chunk
    base = pl.multiple_of(s * rb, 8)
    for k in range(rb // chunk):
        copy(base + k * chunk, chunk, False)
    if ex:
        @pl.when(s == NS - 1)
        def _():
            copy(n - ex, ex, True)


NB = 3


def _deg_sc(dst, n):
    e = dst.shape[0]
    ept = e // NW
    nfull, tail = ept // EC, ept % EC
    assert e % NW == 0 and tail % 8 == 0 and ept % 8 == 0
    assert nfull % NB == 0 and nfull // NB >= 2

    mesh = plsc.VectorSubcoreMesh(core_axis_name="c", subcore_axis_name="s")

    def body(dst_hbm, out_hbm, acc, ones_v, ones_t,
             didx0, didx1, didx2, didx_t, zbuf, isem0, isem1, isem2):
        didx = [didx0, didx1, didx2]
        isem = [isem0, isem1, isem2]
        c = lax.axis_index("c")
        s = lax.axis_index("s")
        wid = s * NC + c

        zero16 = jnp.zeros((16,), jnp.float32)
        one16 = jnp.ones((16,), jnp.float32)

        def fill(i, _):
            zbuf[i, :] = zero16
            return 0

        lax.fori_loop(0, ZB, fill, 0)

        def fill2(i, _):
            ones_v[i, :] = one16
            return 0

        lax.fori_loop(0, EC, fill2, 0)
        if tail:
            def fill3(i, _):
                ones_t[i, :] = one16
                return 0

            lax.fori_loop(0, tail, fill3, 0)

        def zcopy(off, cnt, is_tail):
            zsrc = zbuf.at[pl.ds(0, cnt)] if is_tail else zbuf
            pltpu.sync_copy(zsrc, acc.at[pl.ds(off, cnt)])

        _rows_out(acc, out_hbm, c, s, n, zcopy, ZB)
        plsc.subcore_barrier()

        eb = wid * ept

        def idx_issue(jj, b):
            pltpu.async_copy(dst_hbm.at[pl.ds(eb + jj * EC, EC)],
                             didx[b], isem[b])

        def idx_wait(b):
            pltpu.make_async_copy(dst_hbm.at[pl.ds(0, EC)],
                                  didx[b], isem[b]).wait()

        def step(j, b, issue):
            idx_wait(b)
            pltpu.sync_copy(ones_v, acc.at[didx[b]], add=True)
            if issue:
                idx_issue(j + NB, b)

        for b in range(NB):
            idx_issue(b, b)

        def outer(k, _):
            for b in range(NB):
                step(k * NB + b, b, True)
            return 0

        lax.fori_loop(0, nfull // NB - 1, outer, 0)
        for b in range(NB):
            step(nfull - NB + b, b, False)

        if tail:
            pltpu.sync_copy(dst_hbm.at[pl.ds(eb + nfull * EC, tail)], didx_t)
            pltpu.sync_copy(ones_t, acc.at[didx_t], add=True)

        plsc.subcore_barrier()

        def wcopy(off, cnt, is_tail):
            pltpu.sync_copy(acc.at[pl.ds(off, cnt)],
                            out_hbm.at[c, pl.ds(off, cnt)])

        _rows_out(acc, out_hbm, c, s, n, wcopy, ZC)

    scratch = [
        pltpu.VMEM_SHARED((n, DW), jnp.float32),
        pltpu.VMEM((EC, DW), jnp.float32),
        pltpu.VMEM((max(tail, 8), DW), jnp.float32),
        pltpu.VMEM((EC,), jnp.int32),
        pltpu.VMEM((EC,), jnp.int32),
        pltpu.VMEM((EC,), jnp.int32),
        pltpu.VMEM((max(tail, 8),), jnp.int32),
        pltpu.VMEM((ZB, DW), jnp.float32),
        pltpu.SemaphoreType.DMA,
        pltpu.SemaphoreType.DMA,
        pltpu.SemaphoreType.DMA,
    ]
    return pl.kernel(
        body,
        jax.ShapeDtypeStruct((NC, n, DW), jnp.float32),
        mesh=mesh,
        scratch_types=scratch,
    )(dst)


def _agg_sc(hp, src, dst):
    n, d = hp.shape
    e = src.shape[0]
    ept = e // NW
    nfull, tail = ept // EC, ept % EC
    assert e % NW == 0 and d % 16 == 0
    assert tail % 8 == 0 and ept % 8 == 0
    nba = 2
    assert nfull % nba == 0 and nfull // nba >= 2

    mesh = plsc.VectorSubcoreMesh(core_axis_name="c", subcore_axis_name="s")

    def body(hp_hbm, src_hbm, dst_hbm, out_hbm, acc,
             rows0, rows1, sidx0, sidx1, didx0, didx1,
             rows_t, sidx_t, didx_t, zbuf,
             gsem0, gsem1, isem0, isem1, tsem):
        rows = [rows0, rows1]
        sidx = [sidx0, sidx1]
        didx = [didx0, didx1]
        gsem = [gsem0, gsem1]
        isem = [isem0, isem1]
        c = lax.axis_index("c")
        s = lax.axis_index("s")
        wid = s * NC + c

        zero16 = jnp.zeros((16,), jnp.float32)

        def fill(i, _):
            for q in range(d // 16):
                zbuf[i, pl.ds(q * 16, 16)] = zero16
            return 0

        lax.fori_loop(0, ZB, fill, 0)

        def zcopy(off, cnt, is_tail):
            zsrc = zbuf.at[pl.ds(0, cnt)] if is_tail else zbuf
            pltpu.sync_copy(zsrc, acc.at[pl.ds(off, cnt)])

        _rows_out(acc, out_hbm, c, s, n, zcopy, ZB)
        plsc.subcore_barrier()

        eb = wid * ept

        def idx_issue(jj, b):
            b0 = eb + jj * EC
            pltpu.async_copy(src_hbm.at[pl.ds(b0, EC)], sidx[b], isem[b])
            pltpu.async_copy(dst_hbm.at[pl.ds(b0, EC)], didx[b], isem[b])

        def idx_wait(b):
            pltpu.make_async_copy(src_hbm.at[pl.ds(0, EC)],
                                  sidx[b], isem[b]).wait()
            pltpu.make_async_copy(dst_hbm.at[pl.ds(0, EC)],
                                  didx[b], isem[b]).wait()

        def gather_issue(b):
            pltpu.async_copy(hp_hbm.at[sidx[b]], rows[b], gsem[b])

        def gather_wait(b):
            pltpu.make_async_copy(hp_hbm.at[sidx[b]], rows[b], gsem[b]).wait()

        def step(j, b, issue_idx, issue_gather):
            if issue_gather:
                b1 = (b + 1) % nba
                idx_wait(b1)
                gather_issue(b1)
            gather_wait(b)
            pltpu.sync_copy(rows[b], acc.at[didx[b]], add=True)
            if issue_idx:
                idx_issue(j + nba, b)

        for b in range(nba):
            idx_issue(b, b)
        idx_wait(0)
        gather_issue(0)

        def outer(k, _):
            for b in range(nba):
                step(k * nba + b, b, True, True)
            return 0

        lax.fori_loop(0, nfull // nba - 1, outer, 0)
        for b in range(nba):
            step(nfull - nba + b, b, False, b < nba - 1)

        if tail:
            b0 = eb + nfull * EC
            pltpu.sync_copy(src_hbm.at[pl.ds(b0, tail)], sidx_t)
            pltpu.sync_copy(dst_hbm.at[pl.ds(b0, tail)], didx_t)
            pltpu.async_copy(hp_hbm.at[sidx_t], rows_t, tsem).wait()
            pltpu.sync_copy(rows_t, acc.at[didx_t], add=True)

        plsc.subcore_barrier()

        def wcopy(off, cnt, is_tail):
            pltpu.sync_copy(acc.at[pl.ds(off, cnt)],
                            out_hbm.at[c, pl.ds(off, cnt)])

        _rows_out(acc, out_hbm, c, s, n, wcopy, ZC)

    scratch = [
        pltpu.VMEM_SHARED((n, d), jnp.float32),
        pltpu.VMEM((EC, d), jnp.float32),
        pltpu.VMEM((EC, d), jnp.float32),
        pltpu.VMEM((EC,), jnp.int32),
        pltpu.VMEM((EC,), jnp.int32),
        pltpu.VMEM((EC,), jnp.int32),
        pltpu.VMEM((EC,), jnp.int32),
        pltpu.VMEM((max(tail, 8), d), jnp.float32),
        pltpu.VMEM((max(tail, 8),), jnp.int32),
        pltpu.VMEM((max(tail, 8),), jnp.int32),
        pltpu.VMEM((ZB, d), jnp.float32),
        pltpu.SemaphoreType.DMA,
        pltpu.SemaphoreType.DMA,
        pltpu.SemaphoreType.DMA,
        pltpu.SemaphoreType.DMA,
        pltpu.SemaphoreType.DMA,
    ]
    return pl.kernel(
        body,
        jax.ShapeDtypeStruct((NC, n, d), jnp.float32),
        mesh=mesh,
        scratch_types=scratch,
    )(hp, src, dst)


_TC_R = 1000


def _dinv_of(g):
    deg = g[0, :, :1] + g[1, :, :1] + 1.0
    return lax.rsqrt(deg)


def _tc_first(degp, x, w):
    n, d = x.shape

    def body(g_ref, x_ref, w_ref, o_ref):
        dinv = _dinv_of(g_ref[...])
        o_ref[...] = dinv * jnp.dot(x_ref[...], w_ref[...],
                                    preferred_element_type=jnp.float32)

    return pl.pallas_call(
        body,
        grid=(n // _TC_R,),
        in_specs=[
            pl.BlockSpec((2, _TC_R, DW), lambda i: (0, i, 0)),
            pl.BlockSpec((_TC_R, d), lambda i: (i, 0)),
            pl.BlockSpec((d, w.shape[1]), lambda i: (0, 0)),
        ],
        out_specs=pl.BlockSpec((_TC_R, w.shape[1]), lambda i: (i, 0)),
        out_shape=jax.ShapeDtypeStruct((n, w.shape[1]), jnp.float32),
    )(degp, x, w)


def _tc_mid(degp, tmpp, hp, b, w):
    n, d = hp.shape

    def body(g_ref, t_ref, hp_ref, b_ref, w_ref, o_ref):
        dinv = _dinv_of(g_ref[...])
        t = t_ref[...]
        h = jnp.maximum(dinv * (t[0] + t[1] + hp_ref[...]) + b_ref[...], 0.0)
        o_ref[...] = dinv * jnp.dot(h, w_ref[...],
                                    preferred_element_type=jnp.float32)

    return pl.pallas_call(
        body,
        grid=(n // _TC_R,),
        in_specs=[
            pl.BlockSpec((2, _TC_R, DW), lambda i: (0, i, 0)),
            pl.BlockSpec((2, _TC_R, d), lambda i: (0, i, 0)),
            pl.BlockSpec((_TC_R, d), lambda i: (i, 0)),
            pl.BlockSpec((1, d), lambda i: (0, 0)),
            pl.BlockSpec((d, w.shape[1]), lambda i: (0, 0)),
        ],
        out_specs=pl.BlockSpec((_TC_R, w.shape[1]), lambda i: (i, 0)),
        out_shape=jax.ShapeDtypeStruct((n, w.shape[1]), jnp.float32),
    )(degp, tmpp, hp, b, w)


def _tc_last(degp, tmpp, hp, b, wc, bc):
    n, d = hp.shape
    dout = wc.shape[1]

    def body(g_ref, t_ref, hp_ref, b_ref, w_ref, bc_ref, o_ref):
        dinv = _dinv_of(g_ref[...])
        t = t_ref[...]
        h = jnp.maximum(dinv * (t[0] + t[1] + hp_ref[...]) + b_ref[...], 0.0)
        o_ref[...] = jnp.dot(h, w_ref[...],
                             preferred_element_type=jnp.float32) + bc_ref[...]

    return pl.pallas_call(
        body,
        grid=(n // _TC_R,),
        in_specs=[
            pl.BlockSpec((2, _TC_R, DW), lambda i: (0, i, 0)),
            pl.BlockSpec((2, _TC_R, d), lambda i: (0, i, 0)),
            pl.BlockSpec((_TC_R, d), lambda i: (i, 0)),
            pl.BlockSpec((1, d), lambda i: (0, 0)),
            pl.BlockSpec((d, dout), lambda i: (0, 0)),
            pl.BlockSpec((1, dout), lambda i: (0, 0)),
        ],
        out_specs=pl.BlockSpec((_TC_R, dout), lambda i: (i, 0)),
        out_shape=jax.ShapeDtypeStruct((n, dout), jnp.float32),
    )(degp, tmpp, hp, b, wc, bc)


def kernel(x, edge_index, W1, b1, W2, b2, W3, b3, Wc, bc):
    src = edge_index[0]
    dst = edge_index[1]
    n = x.shape[0]

    degp = _deg_sc(dst, n)
    hp = _tc_first(degp, x, W1)
    for (b, wn) in ((b1, W2), (b2, W3)):
        tmpp = _agg_sc(hp, src, dst)
        hp = _tc_mid(degp, tmpp, hp, b.reshape(1, -1), wn)
    tmpp = _agg_sc(hp, src, dst)
    return _tc_last(degp, tmpp, hp, b3.reshape(1, -1), Wc, bc.reshape(1, -1))


# speedup vs baseline: 29.8446x; 1.1697x over previous
import functools

import jax
import jax.numpy as jnp
from jax import lax
from jax.experimental import pallas as pl
from jax.experimental.pallas import tpu as pltpu
from jax.experimental.pallas import tpu_sc as plsc

NC = 2
NS = 16
NW = NC * NS
EC = 128
DW = 16
ZC = 208
ZB = 48


def _rows_out(acc, out_hbm, c, s, n, copy, chunk):
    rb = (n // NS) & ~7
    ex = n - NS * rb
    assert rb % chunk == 0 and ex % 8 == 0 and ex < chunk
    base = pl.multiple_of(s * rb, 8)
    for k in range(rb // chunk):
        copy(base + k * chunk, chunk, False)
    if ex:
        @pl.when(s == NS - 1)
        def _():
            copy(n - ex, ex, True)


NB = 3


def _deg_sc(dst, n):
    e = dst.shape[0]
    ept = e // NW
    nfull, tail = ept // EC, ept % EC
    assert e % NW == 0 and tail % 8 == 0 and ept % 8 == 0
    assert nfull % NB == 0 and nfull // NB >= 2

    mesh = plsc.VectorSubcoreMesh(core_axis_name="c", subcore_axis_name="s")

    def body(dst_hbm, out_hbm, acc, ones_v, ones_t,
             didx0, didx1, didx2, didx_t, zbuf, isem0, isem1, isem2):
        didx = [didx0, didx1, didx2]
        isem = [isem0, isem1, isem2]
        c = lax.axis_index("c")
        s = lax.axis_index("s")
        wid = s * NC + c

        zero16 = jnp.zeros((16,), jnp.float32)
        one16 = jnp.ones((16,), jnp.float32)

        def fill(i, _):
            zbuf[i, :] = zero16
            return 0

        lax.fori_loop(0, ZB, fill, 0)

        def fill2(i, _):
            ones_v[i, :] = one16
            return 0

        lax.fori_loop(0, EC, fill2, 0)
        if tail:
            def fill3(i, _):
                ones_t[i, :] = one16
                return 0

            lax.fori_loop(0, tail, fill3, 0)

        def zcopy(off, cnt, is_tail):
            zsrc = zbuf.at[pl.ds(0, cnt)] if is_tail else zbuf
            pltpu.sync_copy(zsrc, acc.at[pl.ds(off, cnt)])

        _rows_out(acc, out_hbm, c, s, n, zcopy, ZB)
        plsc.subcore_barrier()

        eb = wid * ept

        def idx_issue(jj, b):
            pltpu.async_copy(dst_hbm.at[pl.ds(eb + jj * EC, EC)],
                             didx[b], isem[b])

        def idx_wait(b):
            pltpu.make_async_copy(dst_hbm.at[pl.ds(0, EC)],
                                  didx[b], isem[b]).wait()

        def step(j, b, issue):
            idx_wait(b)
            pltpu.sync_copy(ones_v, acc.at[didx[b]], add=True)
            if issue:
                idx_issue(j + NB, b)

        for b in range(NB):
            idx_issue(b, b)

        def outer(k, _):
            for b in range(NB):
                step(k * NB + b, b, True)
            return 0

        lax.fori_loop(0, nfull // NB - 1, outer, 0)
        for b in range(NB):
            step(nfull - NB + b, b, False)

        if tail:
            pltpu.sync_copy(dst_hbm.at[pl.ds(eb + nfull * EC, tail)], didx_t)
            pltpu.sync_copy(ones_t, acc.at[didx_t], add=True)

        plsc.subcore_barrier()

        def wcopy(off, cnt, is_tail):
            pltpu.sync_copy(acc.at[pl.ds(off, cnt)],
                            out_hbm.at[c, pl.ds(off, cnt)])

        _rows_out(acc, out_hbm, c, s, n, wcopy, ZC)

    scratch = [
        pltpu.VMEM_SHARED((n, DW), jnp.float32),
        pltpu.VMEM((EC, DW), jnp.float32),
        pltpu.VMEM((max(tail, 8), DW), jnp.float32),
        pltpu.VMEM((EC,), jnp.int32),
        pltpu.VMEM((EC,), jnp.int32),
        pltpu.VMEM((EC,), jnp.int32),
        pltpu.VMEM((max(tail, 8),), jnp.int32),
        pltpu.VMEM((ZB, DW), jnp.float32),
        pltpu.SemaphoreType.DMA,
        pltpu.SemaphoreType.DMA,
        pltpu.SemaphoreType.DMA,
    ]
    return pl.kernel(
        body,
        jax.ShapeDtypeStruct((NC, n, DW), jnp.float32),
        mesh=mesh,
        scratch_types=scratch,
    )(dst)


def _agg_sc(hp, src, dst):
    n, d = hp.shape
    e = src.shape[0]
    ept = e // NW
    nfull, tail = ept // EC, ept % EC
    eca = 80
    nbr = 3
    nbi = 6
    nss = 2
    nfull = ept // eca
    assert e % NW == 0 and d % 16 == 0 and ept % eca == 0 and eca % 8 == 0
    niter = (nfull - 2 - 7) // 6
    peel = list(range(2 + 6 * niter, nfull))
    assert niter >= 1 and all(j + 4 < nfull for j in (0, 1))

    mesh = plsc.VectorSubcoreMesh(core_axis_name="c", subcore_axis_name="s")

    def body(hp_hbm, src_hbm, dst_hbm, out_hbm, acc,
             rows0, rows1, rows2, si0, si1, si2, si3, si4, si5,
             di0, di1, di2, di3, di4, di5, zbuf,
             gs0, gs1, gs2, is0, is1, is2, is3, is4, is5, ss0, ss1):
        rows = [rows0, rows1, rows2]
        sidx = [si0, si1, si2, si3, si4, si5]
        didx = [di0, di1, di2, di3, di4, di5]
        gsem = [gs0, gs1, gs2]
        isem = [is0, is1, is2, is3, is4, is5]
        ssem = [ss0, ss1]
        c = lax.axis_index("c")
        s = lax.axis_index("s")
        wid = s * NC + c

        zero16 = jnp.zeros((16,), jnp.float32)

        def fill(i, _):
            for q in range(d // 16):
                zbuf[i, pl.ds(q * 16, 16)] = zero16
            return 0

        lax.fori_loop(0, ZB, fill, 0)

        def zcopy(off, cnt, is_tail):
            zsrc = zbuf.at[pl.ds(0, cnt)] if is_tail else zbuf
            pltpu.sync_copy(zsrc, acc.at[pl.ds(off, cnt)])

        _rows_out(acc, out_hbm, c, s, n, zcopy, ZB)
        plsc.subcore_barrier()

        eb = wid * ept

        def idx_issue(jj, p):
            b0 = eb + jj * eca
            pltpu.async_copy(src_hbm.at[pl.ds(b0, eca)], sidx[p], isem[p])
            pltpu.async_copy(dst_hbm.at[pl.ds(b0, eca)], didx[p], isem[p])

        def idx_wait(p):
            pltpu.make_async_copy(src_hbm.at[pl.ds(0, eca)],
                                  sidx[p], isem[p]).wait()
            pltpu.make_async_copy(dst_hbm.at[pl.ds(0, eca)],
                                  didx[p], isem[p]).wait()

        def gather_issue(p):
            pltpu.async_copy(hp_hbm.at[sidx[p]], rows[p % nbr],
                             gsem[p % nbr])

        def gather_wait(p):
            pltpu.make_async_copy(hp_hbm.at[sidx[p]], rows[p % nbr],
                                  gsem[p % nbr]).wait()

        def scat_issue(p):
            pltpu.async_copy(rows[p % nbr], acc.at[didx[p]],
                             ssem[p % nss], add=True)

        def scat_wait(ss):
            pltpu.make_async_copy(hp_hbm.at[pl.ds(0, eca)], rows[0],
                                  ssem[ss]).wait()

        def step(j, m, wait_scat=True, iss_idx=True, iss_gather=True):
            if wait_scat:
                scat_wait(m % nss)
            if iss_idx:
                idx_issue(j + 4, (m + 4) % nbi)
            if iss_gather:
                idx_wait((m + 1) % nbi)
                gather_issue((m + 1) % nbi)
            gather_wait(m)
            scat_issue(m)

        for p in range(4):
            idx_issue(p, p)
        idx_wait(0)
        gather_issue(0)
        step(0, 0, wait_scat=False)
        step(1, 1, wait_scat=False)

        def outer(k, _):
            for u in range(6):
                step(2 + k * 6 + u, (2 + u) % 6)
            return 0

        lax.fori_loop(0, niter, outer, 0)
        for j in peel:
            step(j, j % 6, iss_idx=j + 4 < nfull, iss_gather=j + 1 < nfull)
        scat_wait((nfull - 2) % nss)
        scat_wait((nfull - 1) % nss)

        plsc.subcore_barrier()

        def wcopy(off, cnt, is_tail):
            pltpu.sync_copy(acc.at[pl.ds(off, cnt)],
                            out_hbm.at[c, pl.ds(off, cnt)])

        _rows_out(acc, out_hbm, c, s, n, wcopy, ZC)

    scratch = (
        [pltpu.VMEM_SHARED((n, d), jnp.float32)]
        + [pltpu.VMEM((eca, d), jnp.float32)] * 3
        + [pltpu.VMEM((eca,), jnp.int32)] * 12
        + [pltpu.VMEM((ZB, d), jnp.float32)]
        + [pltpu.SemaphoreType.DMA] * 11
    )
    return pl.kernel(
        body,
        jax.ShapeDtypeStruct((NC, n, d), jnp.float32),
        mesh=mesh,
        scratch_types=scratch,
    )(hp, src, dst)


_TC_R = 1000


def _dinv_of(g):
    deg = g[0, :, :1] + g[1, :, :1] + 1.0
    return lax.rsqrt(deg)


def _tc_first(degp, x, w):
    n, d = x.shape

    def body(g_ref, x_ref, w_ref, o_ref):
        dinv = _dinv_of(g_ref[...])
        o_ref[...] = dinv * jnp.dot(x_ref[...], w_ref[...],
                                    preferred_element_type=jnp.float32)

    return pl.pallas_call(
        body,
        grid=(n // _TC_R,),
        in_specs=[
            pl.BlockSpec((2, _TC_R, DW), lambda i: (0, i, 0)),
            pl.BlockSpec((_TC_R, d), lambda i: (i, 0)),
            pl.BlockSpec((d, w.shape[1]), lambda i: (0, 0)),
        ],
        out_specs=pl.BlockSpec((_TC_R, w.shape[1]), lambda i: (i, 0)),
        out_shape=jax.ShapeDtypeStruct((n, w.shape[1]), jnp.float32),
    )(degp, x, w)


def _tc_mid(degp, tmpp, hp, b, w):
    n, d = hp.shape

    def body(g_ref, t_ref, hp_ref, b_ref, w_ref, o_ref):
        dinv = _dinv_of(g_ref[...])
        t = t_ref[...]
        h = jnp.maximum(dinv * (t[0] + t[1] + hp_ref[...]) + b_ref[...], 0.0)
        o_ref[...] = dinv * jnp.dot(h, w_ref[...],
                                    preferred_element_type=jnp.float32)

    return pl.pallas_call(
        body,
        grid=(n // _TC_R,),
        in_specs=[
            pl.BlockSpec((2, _TC_R, DW), lambda i: (0, i, 0)),
            pl.BlockSpec((2, _TC_R, d), lambda i: (0, i, 0)),
            pl.BlockSpec((_TC_R, d), lambda i: (i, 0)),
            pl.BlockSpec((1, d), lambda i: (0, 0)),
            pl.BlockSpec((d, w.shape[1]), lambda i: (0, 0)),
        ],
        out_specs=pl.BlockSpec((_TC_R, w.shape[1]), lambda i: (i, 0)),
        out_shape=jax.ShapeDtypeStruct((n, w.shape[1]), jnp.float32),
    )(degp, tmpp, hp, b, w)


def _tc_last(degp, tmpp, hp, b, wc, bc):
    n, d = hp.shape
    dout = wc.shape[1]

    def body(g_ref, t_ref, hp_ref, b_ref, w_ref, bc_ref, o_ref):
        dinv = _dinv_of(g_ref[...])
        t = t_ref[...]
        h = jnp.maximum(dinv * (t[0] + t[1] + hp_ref[...]) + b_ref[...], 0.0)
        o_ref[...] = jnp.dot(h, w_ref[...],
                             preferred_element_type=jnp.float32) + bc_ref[...]

    return pl.pallas_call(
        body,
        grid=(n // _TC_R,),
        in_specs=[
            pl.BlockSpec((2, _TC_R, DW), lambda i: (0, i, 0)),
            pl.BlockSpec((2, _TC_R, d), lambda i: (0, i, 0)),
            pl.BlockSpec((_TC_R, d), lambda i: (i, 0)),
            pl.BlockSpec((1, d), lambda i: (0, 0)),
            pl.BlockSpec((d, dout), lambda i: (0, 0)),
            pl.BlockSpec((1, dout), lambda i: (0, 0)),
        ],
        out_specs=pl.BlockSpec((_TC_R, dout), lambda i: (i, 0)),
        out_shape=jax.ShapeDtypeStruct((n, dout), jnp.float32),
    )(degp, tmpp, hp, b, wc, bc)


def kernel(x, edge_index, W1, b1, W2, b2, W3, b3, Wc, bc):
    src = edge_index[0]
    dst = edge_index[1]
    n = x.shape[0]

    degp = _deg_sc(dst, n)
    hp = _tc_first(degp, x, W1)
    for (b, wn) in ((b1, W2), (b2, W3)):
        tmpp = _agg_sc(hp, src, dst)
        hp = _tc_mid(degp, tmpp, hp, b.reshape(1, -1), wn)
    tmpp = _agg_sc(hp, src, dst)
    return _tc_last(degp, tmpp, hp, b3.reshape(1, -1), Wc, bc.reshape(1, -1))
